# baseline (speedup 1.0000x reference)
import functools

import jax
import jax.numpy as jnp
from jax import lax
from jax.experimental import pallas as pl
from jax.experimental.pallas import tpu as pltpu
from jax.experimental.pallas import tpu_sc as plsc

B = 16
L_PAD = 1024
D = 1024
HID = 16384
TOPK = 32
NUM_VIEWS = 8
GAMMA = 10.0
EPS = 1e-6
T_LEN = 256

ENC_PREC = lax.Precision.DEFAULT
DEC_PREC = lax.Precision.DEFAULT

_INTERPRET = False


def _clipnorm(x, axis):
    n = jnp.sqrt(jnp.sum(x * x, axis=axis, keepdims=True))
    return x / jnp.clip(n, 1e-12)


def _prep_kernel(gt_ref, tp_ref, tm_ref, v0_ref, cx_ref, cy_ref,
                 tg_ref, xnt_ref, vv_ref, xnv_ref):
    tm = tm_ref[...]
    tp = tp_ref[...]
    ts = jnp.sum(tp * tm[:, :, None], axis=1)
    tg = ts / (jnp.sum(tm, axis=1, keepdims=True) + 1e-6)
    tg_ref[...] = tg
    xnt_ref[...] = _clipnorm(tg, -1)

    hg = gt_ref[1].astype(jnp.float32)
    wg = gt_ref[2].astype(jnp.float32)
    x0 = 0.5 / wg
    y0 = 0.5 / hg
    cx = cx_ref[...]
    cy = cy_ref[...]
    dist = (cx - x0) ** 2 + (cy - y0) ** 2
    m = jnp.exp(-GAMMA * dist)
    v0 = v0_ref[...]
    num = m[:, :, None] * v0[:, None, :]
    vv = num / (m + EPS)[:, :, None]
    vv_ref[...] = vv
    xnv_ref[...] = _clipnorm(vv, -1)


def _enc_kernel(x_ref, e_ref, o_ref):
    e = e_ref[...]
    n2 = jnp.sum(e * e, axis=1, keepdims=True)
    w = e / jnp.clip(jnp.sqrt(n2), 1e-12)
    raw = lax.dot_general(x_ref[...], w, (((1,), (1,)), ((), ())),
                          precision=ENC_PREC)
    cos = jnp.clip(raw, -1.0, 1.0)
    o_ref[...] = 2.0 - jnp.sqrt(2.0 - 2.0 * cos)


_NCHUNK = HID // 16
_CBUF = 256
_CAP = 224


def _sc_process_row(src, dst, row_v, zbuf_v, cbv_v, cbi_v):
    pltpu.sync_copy(src, row_v)

    def _mom(i, carry):
        s, mx = carry
        v = row_v[pl.ds(i * 16, 16)]
        return s + v, jnp.maximum(mx, v)

    s, mx = lax.fori_loop(
        0, _NCHUNK, _mom,
        (jnp.zeros((16,), jnp.float32), jnp.full((16,), -1.0, jnp.float32)))
    mu = jnp.sum(s) * (1.0 / HID)
    big = jnp.max(mx)

    neg1 = jnp.full((16,), -1.0, jnp.float32)

    def _collect(theta):
        for j in range(_CBUF // 16):
            cbv_v[pl.ds(j * 16, 16)] = neg1

        def _cbody(i, cnt):
            v = row_v[pl.ds(i * 16, 16)]
            m = v > theta
            c = jnp.sum(m.astype(jnp.int32))

            @pl.when((c > 0) & (cnt <= _CAP))
            def _():
                idx = lax.iota(jnp.int32, 16) + i * 16
                plsc.store_compressed(cbv_v.at[pl.ds(cnt, 16)], v, mask=m)
                plsc.store_compressed(cbi_v.at[pl.ds(cnt, 16)], idx, mask=m)

            return cnt + c

        return lax.fori_loop(0, _NCHUNK, _cbody, jnp.int32(0))

    def _wcond(carry):
        _, _, _, cnt, it = carry
        return ((cnt < TOPK) | (cnt > _CAP)) & (it < 10)

    def _wbody(carry):
        tlo, thi, theta, _, it = carry
        cnt = _collect(theta)
        tlo = jnp.where(cnt > _CAP, theta, tlo)
        thi = jnp.where(cnt < TOPK, theta, thi)
        return tlo, thi, 0.5 * (tlo + thi), cnt, it + 1

    theta0 = big - 0.26 * (big - mu)
    _, _, _, cnt, _ = lax.while_loop(
        _wcond, _wbody,
        (mu - (big - mu), big, theta0, jnp.int32(-1), jnp.int32(0)))
    success = (cnt >= TOPK) & (cnt <= _CAP)

    @pl.when(success)
    def _small():
        vals = [cbv_v[pl.ds(j * 16, 16)] for j in range(_CBUF // 16)]
        idxs = [cbi_v[pl.ds(j * 16, 16)] for j in range(_CBUF // 16)]
        bits = [plsc.bitcast(v, jnp.int32) for v in vals]

        def _vbit(i, tau):
            cand = tau | lax.shift_left(jnp.int32(1), 30 - i)
            acc = jnp.zeros((16,), jnp.int32)
            for b in bits:
                acc = acc + (b >= cand).astype(jnp.int32)
            return jnp.where(jnp.sum(acc) >= TOPK, cand, tau)

        tau = lax.fori_loop(0, 31, _vbit, jnp.int32(0))

        accm = jnp.zeros((16,), jnp.int32)
        for b in bits:
            accm = accm + (b > tau).astype(jnp.int32)
        need = TOPK - jnp.sum(accm)

        def _ibit(i, c):
            cand = c | lax.shift_left(jnp.int32(1), 13 - i)
            acc = jnp.zeros((16,), jnp.int32)
            for b, ix in zip(bits, idxs):
                acc = acc + ((b == tau) & (ix < cand)).astype(jnp.int32)
            return jnp.where(jnp.sum(acc) < need, cand, c)

        cidx = lax.fori_loop(0, 14, _ibit, jnp.int32(0))

        zero16 = jnp.zeros((16,), jnp.float32)
        sels = []
        for b, ix, v in zip(bits, idxs, vals):
            sel = (b > tau) | ((b == tau) & (ix <= cidx))
            sels.append(sel)
            plsc.store_scatter(zbuf_v, [ix], v, mask=sel)
        pltpu.sync_copy(zbuf_v, dst)
        for ix, sel in zip(idxs, sels):
            plsc.store_scatter(zbuf_v, [ix], zero16, mask=sel)

    @pl.when(jnp.logical_not(success))
    def _full():
        def _count_ge(cand):
            def _b(i, cnt):
                v = row_v[pl.ds(i * 16, 16)]
                b = plsc.bitcast(v, jnp.int32)
                return cnt + jnp.sum((b >= cand).astype(jnp.int32))
            return lax.fori_loop(0, _NCHUNK, _b, jnp.int32(0))

        def _vbit(i, tau):
            cand = tau | lax.shift_left(jnp.int32(1), 30 - i)
            return jnp.where(_count_ge(cand) >= TOPK, cand, tau)

        tau = lax.fori_loop(0, 31, _vbit, jnp.int32(0))
        need = TOPK - _count_ge(tau + 1)

        def _ibit(i, c):
            cand = c | lax.shift_left(jnp.int32(1), 13 - i)

            def _b(k, cnt):
                v = row_v[pl.ds(k * 16, 16)]
                b = plsc.bitcast(v, jnp.int32)
                ix = lax.iota(jnp.int32, 16) + k * 16
                return cnt + jnp.sum(((b == tau) & (ix < cand)).astype(jnp.int32))

            return jnp.where(lax.fori_loop(0, _NCHUNK, _b, jnp.int32(0)) < need,
                             cand, c)

        cidx = lax.fori_loop(0, 14, _ibit, jnp.int32(0))

        def _w(k, _):
            v = row_v[pl.ds(k * 16, 16)]
            b = plsc.bitcast(v, jnp.int32)
            ix = lax.iota(jnp.int32, 16) + k * 16
            sel = (b > tau) | ((b == tau) & (ix <= cidx))
            zbuf_v[pl.ds(k * 16, 16)] = jnp.where(sel, v, 0.0)
            return 0

        lax.fori_loop(0, _NCHUNK, _w, 0)
        pltpu.sync_copy(zbuf_v, dst)

        def _z(k, _):
            zbuf_v[pl.ds(k * 16, 16)] = jnp.zeros((16,), jnp.float32)
            return 0

        lax.fori_loop(0, _NCHUNK, _z, 0)


def _sc_topk_body(av, at, lv, lt, row_v, zbuf_v, cbv_v, cbi_v):
    nc = 2
    wid = lax.axis_index("s") * nc + lax.axis_index("c")

    def _z(k, _):
        zbuf_v[pl.ds(k * 16, 16)] = jnp.zeros((16,), jnp.float32)
        return 0

    lax.fori_loop(0, _NCHUNK, _z, 0)

    def _vrow(rep, _):
        r = wid * 4 + rep
        _sc_process_row(av.at[r], lv.at[r], row_v, zbuf_v, cbv_v, cbi_v)
        return 0

    lax.fori_loop(0, 4, _vrow, 0)

    @pl.when(wid < 16)
    def _():
        _sc_process_row(at.at[wid], lt.at[wid], row_v, zbuf_v, cbv_v, cbi_v)


def _dec_kernel(l_ref, d_ref, b_ref, o_ref):
    @pl.when(pl.program_id(0) == 0)
    def _init():
        o_ref[...] = jnp.broadcast_to(b_ref[...], o_ref.shape)
    o_ref[...] += lax.dot_general(l_ref[...], d_ref[...],
                                  (((1,), (1,)), ((), ())),
                                  precision=DEC_PREC)


def _encode(x, enc, hb):
    r = x.shape[0]
    return pl.pallas_call(
        _enc_kernel,
        grid=(HID // hb,),
        in_specs=[
            pl.BlockSpec((r, D), lambda h: (0, 0)),
            pl.BlockSpec((hb, D), lambda h: (h, 0)),
        ],
        out_specs=pl.BlockSpec((r, hb), lambda h: (0, h)),
        out_shape=jax.ShapeDtypeStruct((r, HID), jnp.float32),
        interpret=_INTERPRET,
    )(x, enc)


def _topk_latent_sc(acts_v, acts_t):
    mesh = plsc.VectorSubcoreMesh(core_axis_name="c", subcore_axis_name="s",
                                  num_cores=2)
    f = pl.kernel(
        _sc_topk_body,
        mesh=mesh,
        compiler_params=pltpu.CompilerParams(needs_layout_passes=False),
        out_type=[
            jax.ShapeDtypeStruct((B * NUM_VIEWS, HID), jnp.float32),
            jax.ShapeDtypeStruct((B, HID), jnp.float32),
        ],
        scratch_types=[
            pltpu.VMEM((HID,), jnp.float32),
            pltpu.VMEM((HID,), jnp.float32),
            pltpu.VMEM((_CBUF,), jnp.float32),
            pltpu.VMEM((_CBUF,), jnp.int32),
        ],
    )
    return f(acts_v, acts_t)


def _decode(latent, dec_w, dec_b, hb):
    r = latent.shape[0]
    return pl.pallas_call(
        _dec_kernel,
        grid=(HID // hb,),
        in_specs=[
            pl.BlockSpec((r, hb), lambda h: (0, h)),
            pl.BlockSpec((D, hb), lambda h: (0, h)),
            pl.BlockSpec((1, D), lambda h: (0, 0)),
        ],
        out_specs=pl.BlockSpec((r, D), lambda h: (0, 0)),
        out_shape=jax.ShapeDtypeStruct((r, D), jnp.float32),
        interpret=_INTERPRET,
    )(latent, dec_w, dec_b.reshape(1, D))


def kernel(v_pad, v_len, grid_thws, t_pad, t_mask, centers,
           encoder_v, decoder_v_w, decoder_v_b,
           encoder_t, decoder_t_w, decoder_t_b):
    del v_len
    v0 = v_pad[:, 0, :]
    cx = centers[:, :, 0]
    cy = centers[:, :, 1]
    gt = grid_thws[0]

    t_global, xn_t, v_views, xn_v = pl.pallas_call(
        _prep_kernel,
        in_specs=[
            pl.BlockSpec(memory_space=pltpu.SMEM),
            pl.BlockSpec((B, T_LEN, D), lambda: (0, 0, 0)),
            pl.BlockSpec((B, T_LEN), lambda: (0, 0)),
            pl.BlockSpec((B, D), lambda: (0, 0)),
            pl.BlockSpec((B, NUM_VIEWS), lambda: (0, 0)),
            pl.BlockSpec((B, NUM_VIEWS), lambda: (0, 0)),
        ],
        out_specs=[
            pl.BlockSpec((B, D), lambda: (0, 0)),
            pl.BlockSpec((B, D), lambda: (0, 0)),
            pl.BlockSpec((B, NUM_VIEWS, D), lambda: (0, 0, 0)),
            pl.BlockSpec((B, NUM_VIEWS, D), lambda: (0, 0, 0)),
        ],
        out_shape=[
            jax.ShapeDtypeStruct((B, D), jnp.float32),
            jax.ShapeDtypeStruct((B, D), jnp.float32),
            jax.ShapeDtypeStruct((B, NUM_VIEWS, D), jnp.float32),
            jax.ShapeDtypeStruct((B, NUM_VIEWS, D), jnp.float32),
        ],
        interpret=_INTERPRET,
    )(gt, t_pad, t_mask, v0, cx, cy)

    xv = xn_v.reshape(B * NUM_VIEWS, D)

    acts_v = _encode(xv, encoder_v, 2048)
    acts_t = _encode(xn_t, encoder_t, 2048)

    latent_v, latent_t = _topk_latent_sc(acts_v, acts_t)

    recon_v = _decode(latent_v, decoder_v_w, decoder_v_b, 2048)
    recon_t = _decode(latent_t, decoder_t_w, decoder_t_b, 2048)

    return (recon_v.reshape(B, NUM_VIEWS, D), v_views, recon_t, t_global,
            latent_v.reshape(B, NUM_VIEWS, HID), latent_t)


# speedup vs baseline: 3.7447x; 3.7447x over previous
import functools

import jax
import jax.numpy as jnp
from jax import lax
from jax.experimental import pallas as pl
from jax.experimental.pallas import tpu as pltpu

B = 16
L_PAD = 1024
D = 1024
HID = 16384
TOPK = 32
NUM_VIEWS = 8
GAMMA = 10.0
EPS = 1e-6
T_LEN = 256

ENC_PREC = lax.Precision.DEFAULT
DEC_PREC = lax.Precision.DEFAULT

_INTERPRET = False


def _clipnorm(x, axis):
    n = jnp.sqrt(jnp.sum(x * x, axis=axis, keepdims=True))
    return x / jnp.clip(n, 1e-12)


def _prep_kernel(gt_ref, tp_ref, tm_ref, v0_ref, cx_ref, cy_ref,
                 tg_ref, xnt_ref, vv_ref, xnv_ref):
    tm = tm_ref[...]
    tp = tp_ref[...]
    ts = jnp.sum(tp * tm[:, :, None], axis=1)
    tg = ts / (jnp.sum(tm, axis=1, keepdims=True) + 1e-6)
    tg_ref[...] = tg
    xnt_ref[...] = _clipnorm(tg, -1)

    hg = gt_ref[1].astype(jnp.float32)
    wg = gt_ref[2].astype(jnp.float32)
    x0 = 0.5 / wg
    y0 = 0.5 / hg
    cx = cx_ref[...]
    cy = cy_ref[...]
    dist = (cx - x0) ** 2 + (cy - y0) ** 2
    m = jnp.exp(-GAMMA * dist)
    v0 = v0_ref[...]
    num = m[:, :, None] * v0[:, None, :]
    vv = num / (m + EPS)[:, :, None]
    vv_ref[...] = vv
    xnv_ref[...] = _clipnorm(vv, -1)


def _enc_kernel(x_ref, e_ref, o_ref):
    e = e_ref[...]
    n2 = jnp.sum(e * e, axis=1, keepdims=True)
    w = e / jnp.clip(jnp.sqrt(n2), 1e-12)
    raw = lax.dot_general(x_ref[...], w, (((1,), (1,)), ((), ())),
                          precision=ENC_PREC)
    cos = jnp.clip(raw, -1.0, 1.0)
    o_ref[...] = 2.0 - jnp.sqrt(2.0 - 2.0 * cos)


def _topk_kernel(a_ref, o_ref, s_ref, *, k):
    a = a_ref[...]
    bits = lax.bitcast_convert_type(a, jnp.int32)
    rb = a.shape[0]
    tau = jnp.zeros((rb, 1), jnp.int32)
    for bit in range(30, -1, -1):
        cand = tau | (1 << bit)
        cnt = jnp.sum((bits >= cand).astype(jnp.int32), axis=1, keepdims=True)
        tau = jnp.where(cnt >= k, cand, tau)
    gt = bits > tau
    m = jnp.sum(gt.astype(jnp.int32), axis=1, keepdims=True)
    eq = bits == tau
    need = k - m
    iota = lax.broadcasted_iota(jnp.int32, a.shape, 1)
    c = jnp.zeros((rb, 1), jnp.int32)
    for bit in range(13, -1, -1):
        cand = c | (1 << bit)
        cnt = jnp.sum((eq & (iota < cand)).astype(jnp.int32),
                      axis=1, keepdims=True)
        c = jnp.where(cnt < need, cand, c)
    sel = gt | (eq & (iota <= c))
    o_ref[...] = jnp.where(sel, a, 0.0)
    s_ref[...] = sel.astype(jnp.float32)


def _topk_latent(acts, rb):
    r = acts.shape[0]
    return pl.pallas_call(
        functools.partial(_topk_kernel, k=TOPK),
        grid=(r // rb,),
        in_specs=[pl.BlockSpec((rb, HID), lambda i: (i, 0))],
        out_specs=[pl.BlockSpec((rb, HID), lambda i: (i, 0)),
                   pl.BlockSpec((rb, HID), lambda i: (i, 0))],
        out_shape=[jax.ShapeDtypeStruct((r, HID), jnp.float32),
                   jax.ShapeDtypeStruct((r, HID), jnp.float32)],
        interpret=_INTERPRET,
    )(acts)


def _verify_kernel(a_ref, s_ref, l_ref, ok_ref):
    a = a_ref[...]
    s = s_ref[...][0, 0] > 0.5
    sb = jnp.broadcast_to(s[None, :], a.shape)
    m1 = jnp.min(jnp.where(sb, a, 3.0), axis=1, keepdims=True)
    m2 = jnp.max(jnp.where(sb, -1.0, a), axis=1, keepdims=True)
    ok = jnp.all(m2 < m1)
    ok_ref[...] = ok.astype(jnp.float32).reshape(1, 1, 1)
    l_ref[...] = jnp.where(sb, a, 0.0)


def _latent_v_fast(acts_v, sel0):
    sel3 = sel0.reshape(B, 1, HID)
    return pl.pallas_call(
        _verify_kernel,
        grid=(B,),
        in_specs=[
            pl.BlockSpec((NUM_VIEWS, HID), lambda i: (i, 0)),
            pl.BlockSpec((1, 1, HID), lambda i: (i, 0, 0)),
        ],
        out_specs=[
            pl.BlockSpec((NUM_VIEWS, HID), lambda i: (i, 0)),
            pl.BlockSpec((1, 1, 1), lambda i: (i, 0, 0)),
        ],
        out_shape=[
            jax.ShapeDtypeStruct((B * NUM_VIEWS, HID), jnp.float32),
            jax.ShapeDtypeStruct((B, 1, 1), jnp.float32),
        ],
        interpret=_INTERPRET,
    )(acts_v, sel3)


def _dec_kernel(l_ref, d_ref, b_ref, o_ref):
    @pl.when(pl.program_id(0) == 0)
    def _init():
        o_ref[...] = jnp.broadcast_to(b_ref[...], o_ref.shape)
    o_ref[...] += lax.dot_general(l_ref[...], d_ref[...],
                                  (((1,), (1,)), ((), ())),
                                  precision=DEC_PREC)


def _encode(x, enc, hb):
    r = x.shape[0]
    return pl.pallas_call(
        _enc_kernel,
        grid=(HID // hb,),
        in_specs=[
            pl.BlockSpec((r, D), lambda h: (0, 0)),
            pl.BlockSpec((hb, D), lambda h: (h, 0)),
        ],
        out_specs=pl.BlockSpec((r, hb), lambda h: (0, h)),
        out_shape=jax.ShapeDtypeStruct((r, HID), jnp.float32),
        interpret=_INTERPRET,
    )(x, enc)


def _decode(latent, dec_w, dec_b, hb):
    r = latent.shape[0]
    return pl.pallas_call(
        _dec_kernel,
        grid=(HID // hb,),
        in_specs=[
            pl.BlockSpec((r, hb), lambda h: (0, h)),
            pl.BlockSpec((D, hb), lambda h: (0, h)),
            pl.BlockSpec((1, D), lambda h: (0, 0)),
        ],
        out_specs=pl.BlockSpec((r, D), lambda h: (0, 0)),
        out_shape=jax.ShapeDtypeStruct((r, D), jnp.float32),
        interpret=_INTERPRET,
    )(latent, dec_w, dec_b.reshape(1, D))


def kernel(v_pad, v_len, grid_thws, t_pad, t_mask, centers,
           encoder_v, decoder_v_w, decoder_v_b,
           encoder_t, decoder_t_w, decoder_t_b):
    del v_len
    v0 = v_pad[:, 0, :]
    cx = centers[:, :, 0]
    cy = centers[:, :, 1]
    gt = grid_thws[0]

    t_global, xn_t, v_views, xn_v = pl.pallas_call(
        _prep_kernel,
        in_specs=[
            pl.BlockSpec(memory_space=pltpu.SMEM),
            pl.BlockSpec((B, T_LEN, D), lambda: (0, 0, 0)),
            pl.BlockSpec((B, T_LEN), lambda: (0, 0)),
            pl.BlockSpec((B, D), lambda: (0, 0)),
            pl.BlockSpec((B, NUM_VIEWS), lambda: (0, 0)),
            pl.BlockSpec((B, NUM_VIEWS), lambda: (0, 0)),
        ],
        out_specs=[
            pl.BlockSpec((B, D), lambda: (0, 0)),
            pl.BlockSpec((B, D), lambda: (0, 0)),
            pl.BlockSpec((B, NUM_VIEWS, D), lambda: (0, 0, 0)),
            pl.BlockSpec((B, NUM_VIEWS, D), lambda: (0, 0, 0)),
        ],
        out_shape=[
            jax.ShapeDtypeStruct((B, D), jnp.float32),
            jax.ShapeDtypeStruct((B, D), jnp.float32),
            jax.ShapeDtypeStruct((B, NUM_VIEWS, D), jnp.float32),
            jax.ShapeDtypeStruct((B, NUM_VIEWS, D), jnp.float32),
        ],
        interpret=_INTERPRET,
    )(gt, t_pad, t_mask, v0, cx, cy)

    xv = xn_v.reshape(B * NUM_VIEWS, D)

    acts_v = _encode(xv, encoder_v, 2048)
    acts_t = _encode(xn_t, encoder_t, 2048)

    a0 = acts_v.reshape(B, NUM_VIEWS, HID)[:, 0]
    _, sel0 = _topk_latent(a0, 8)
    latent_t, _ = _topk_latent(acts_t, 8)
    lat_fast, oks = _latent_v_fast(acts_v, sel0)
    latent_v = lax.cond(jnp.all(oks > 0.5),
                        lambda: lat_fast,
                        lambda: _topk_latent(acts_v, 8)[0])

    recon_v = _decode(latent_v, decoder_v_w, decoder_v_b, 2048)
    recon_t = _decode(latent_t, decoder_t_w, decoder_t_b, 2048)

    return (recon_v.reshape(B, NUM_VIEWS, D), v_views, recon_t, t_global,
            latent_v.reshape(B, NUM_VIEWS, HID), latent_t)


# speedup vs baseline: 4.0262x; 1.0752x over previous
import functools

import jax
import jax.numpy as jnp
from jax import lax
from jax.experimental import pallas as pl
from jax.experimental.pallas import tpu as pltpu

B = 16
L_PAD = 1024
D = 1024
HID = 16384
TOPK = 32
NUM_VIEWS = 8
GAMMA = 10.0
EPS = 1e-6
T_LEN = 256

ENC_PREC = lax.Precision.DEFAULT
DEC_PREC = lax.Precision.DEFAULT

_INTERPRET = False


def _clipnorm(x, axis):
    n = jnp.sqrt(jnp.sum(x * x, axis=axis, keepdims=True))
    return x / jnp.clip(n, 1e-12)


def _prep_kernel(gt_ref, tp_ref, tm_ref, v0_ref, cx_ref, cy_ref,
                 tg_ref, xnt_ref, vv_ref, xnv_ref):
    tm = tm_ref[...]
    tp = tp_ref[...]
    ts = jnp.sum(tp * tm[:, :, None], axis=1)
    tg = ts / (jnp.sum(tm, axis=1, keepdims=True) + 1e-6)
    tg_ref[...] = tg
    xnt_ref[...] = _clipnorm(tg, -1)

    hg = gt_ref[1].astype(jnp.float32)
    wg = gt_ref[2].astype(jnp.float32)
    x0 = 0.5 / wg
    y0 = 0.5 / hg
    cx = cx_ref[...]
    cy = cy_ref[...]
    dist = (cx - x0) ** 2 + (cy - y0) ** 2
    m = jnp.exp(-GAMMA * dist)
    v0 = v0_ref[...]
    num = m[:, :, None] * v0[:, None, :]
    vv = num / (m + EPS)[:, :, None]
    vv_ref[...] = vv
    xnv_ref[...] = _clipnorm(vv, -1)


def _enc_kernel(x_ref, e_ref, o_ref):
    e = e_ref[...]
    n2 = jnp.sum(e * e, axis=1, keepdims=True)
    w = e / jnp.clip(jnp.sqrt(n2), 1e-12)
    raw = lax.dot_general(x_ref[...], w, (((1,), (1,)), ((), ())),
                          precision=ENC_PREC)
    cos = jnp.clip(raw, -1.0, 1.0)
    o_ref[...] = 2.0 - jnp.sqrt(2.0 - 2.0 * cos)


def _topk_kernel(a_ref, o_ref, s_ref, *, k):
    a = a_ref[...]
    bits = lax.bitcast_convert_type(a, jnp.int32)
    rb = a.shape[0]
    tau = jnp.zeros((rb, 1), jnp.int32)
    for bit in range(30, -1, -1):
        cand = tau | (1 << bit)
        cnt = jnp.sum((bits >= cand).astype(jnp.int32), axis=1, keepdims=True)
        tau = jnp.where(cnt >= k, cand, tau)
    gt = bits > tau
    m = jnp.sum(gt.astype(jnp.int32), axis=1, keepdims=True)
    eq = bits == tau
    need = k - m
    iota = lax.broadcasted_iota(jnp.int32, a.shape, 1)
    c = jnp.zeros((rb, 1), jnp.int32)
    for bit in range(13, -1, -1):
        cand = c | (1 << bit)
        cnt = jnp.sum((eq & (iota < cand)).astype(jnp.int32),
                      axis=1, keepdims=True)
        c = jnp.where(cnt < need, cand, c)
    sel = gt | (eq & (iota <= c))
    o_ref[...] = jnp.where(sel, a, 0.0)
    s_ref[...] = sel.astype(jnp.float32)


def _topk_latent(acts, rb):
    r = acts.shape[0]
    return pl.pallas_call(
        functools.partial(_topk_kernel, k=TOPK),
        grid=(r // rb,),
        in_specs=[pl.BlockSpec((rb, HID), lambda i: (i, 0))],
        out_specs=[pl.BlockSpec((rb, HID), lambda i: (i, 0)),
                   pl.BlockSpec((rb, HID), lambda i: (i, 0))],
        out_shape=[jax.ShapeDtypeStruct((r, HID), jnp.float32),
                   jax.ShapeDtypeStruct((r, HID), jnp.float32)],
        interpret=_INTERPRET,
    )(acts)


def _verify_kernel(a_ref, s_ref, l_ref, ok_ref):
    a = a_ref[...]
    s = s_ref[...][0, 0] > 0.5
    sb = jnp.broadcast_to(s[None, :], a.shape)
    m1 = jnp.min(jnp.where(sb, a, 3.0), axis=1, keepdims=True)
    m2 = jnp.max(jnp.where(sb, -1.0, a), axis=1, keepdims=True)
    ok = jnp.all(m2 < m1)
    ok_ref[...] = ok.astype(jnp.float32).reshape(1, 1, 1)
    l_ref[...] = jnp.where(sb, a, 0.0)


def _latent_v_fast(acts_v, sel0):
    sel3 = sel0.reshape(B, 1, HID)
    return pl.pallas_call(
        _verify_kernel,
        grid=(B,),
        in_specs=[
            pl.BlockSpec((NUM_VIEWS, HID), lambda i: (i, 0)),
            pl.BlockSpec((1, 1, HID), lambda i: (i, 0, 0)),
        ],
        out_specs=[
            pl.BlockSpec((NUM_VIEWS, HID), lambda i: (i, 0)),
            pl.BlockSpec((1, 1, 1), lambda i: (i, 0, 0)),
        ],
        out_shape=[
            jax.ShapeDtypeStruct((B * NUM_VIEWS, HID), jnp.float32),
            jax.ShapeDtypeStruct((B, 1, 1), jnp.float32),
        ],
        interpret=_INTERPRET,
    )(acts_v, sel3)


def _dec_kernel(l_ref, d_ref, b_ref, o_ref):
    @pl.when(pl.program_id(0) == 0)
    def _init():
        o_ref[...] = jnp.broadcast_to(b_ref[...], o_ref.shape)
    o_ref[...] += lax.dot_general(l_ref[...], d_ref[...],
                                  (((1,), (1,)), ((), ())),
                                  precision=DEC_PREC)


def _decv_kernel(a_ref, s_ref, d_ref, b_ref, o_ref, l_ref, m1_ref, m2_ref):
    a = a_ref[...]
    s3 = s_ref[...]
    sb = jnp.broadcast_to(s3 > 0.5, (B, NUM_VIEWS, a.shape[1]))
    sb = sb.reshape(B * NUM_VIEWS, a.shape[1])
    lat = jnp.where(sb, a, 0.0)
    l_ref[...] = lat
    m1 = jnp.min(jnp.where(sb, a, 3.0), axis=1, keepdims=True)
    m2 = jnp.max(jnp.where(sb, -1.0, a), axis=1, keepdims=True)

    @pl.when(pl.program_id(0) == 0)
    def _init():
        o_ref[...] = jnp.broadcast_to(b_ref[...], o_ref.shape)
        m1_ref[...] = m1
        m2_ref[...] = m2

    @pl.when(pl.program_id(0) != 0)
    def _acc():
        m1_ref[...] = jnp.minimum(m1_ref[...], m1)
        m2_ref[...] = jnp.maximum(m2_ref[...], m2)

    o_ref[...] += lax.dot_general(lat, d_ref[...],
                                  (((1,), (1,)), ((), ())),
                                  precision=DEC_PREC)


def _decode_v_fused(acts_v, sel0, dec_w, dec_b, hb):
    r = B * NUM_VIEWS
    sel3 = sel0.reshape(B, 1, HID)
    return pl.pallas_call(
        _decv_kernel,
        grid=(HID // hb,),
        in_specs=[
            pl.BlockSpec((r, hb), lambda h: (0, h)),
            pl.BlockSpec((B, 1, hb), lambda h: (0, 0, h)),
            pl.BlockSpec((D, hb), lambda h: (0, h)),
            pl.BlockSpec((1, D), lambda h: (0, 0)),
        ],
        out_specs=[
            pl.BlockSpec((r, D), lambda h: (0, 0)),
            pl.BlockSpec((r, hb), lambda h: (0, h)),
            pl.BlockSpec((r, 1), lambda h: (0, 0)),
            pl.BlockSpec((r, 1), lambda h: (0, 0)),
        ],
        out_shape=[
            jax.ShapeDtypeStruct((r, D), jnp.float32),
            jax.ShapeDtypeStruct((r, HID), jnp.float32),
            jax.ShapeDtypeStruct((r, 1), jnp.float32),
            jax.ShapeDtypeStruct((r, 1), jnp.float32),
        ],
        interpret=_INTERPRET,
    )(acts_v, sel3, dec_w, dec_b.reshape(1, D))


def _encode(x, enc, hb):
    r = x.shape[0]
    return pl.pallas_call(
        _enc_kernel,
        grid=(HID // hb,),
        in_specs=[
            pl.BlockSpec((r, D), lambda h: (0, 0)),
            pl.BlockSpec((hb, D), lambda h: (h, 0)),
        ],
        out_specs=pl.BlockSpec((r, hb), lambda h: (0, h)),
        out_shape=jax.ShapeDtypeStruct((r, HID), jnp.float32),
        interpret=_INTERPRET,
    )(x, enc)


def _decode(latent, dec_w, dec_b, hb):
    r = latent.shape[0]
    return pl.pallas_call(
        _dec_kernel,
        grid=(HID // hb,),
        in_specs=[
            pl.BlockSpec((r, hb), lambda h: (0, h)),
            pl.BlockSpec((D, hb), lambda h: (0, h)),
            pl.BlockSpec((1, D), lambda h: (0, 0)),
        ],
        out_specs=pl.BlockSpec((r, D), lambda h: (0, 0)),
        out_shape=jax.ShapeDtypeStruct((r, D), jnp.float32),
        interpret=_INTERPRET,
    )(latent, dec_w, dec_b.reshape(1, D))


def kernel(v_pad, v_len, grid_thws, t_pad, t_mask, centers,
           encoder_v, decoder_v_w, decoder_v_b,
           encoder_t, decoder_t_w, decoder_t_b):
    del v_len
    v0 = v_pad[:, 0, :]
    cx = centers[:, :, 0]
    cy = centers[:, :, 1]
    gt = grid_thws[0]

    t_global, xn_t, v_views, xn_v = pl.pallas_call(
        _prep_kernel,
        in_specs=[
            pl.BlockSpec(memory_space=pltpu.SMEM),
            pl.BlockSpec((B, T_LEN, D), lambda: (0, 0, 0)),
            pl.BlockSpec((B, T_LEN), lambda: (0, 0)),
            pl.BlockSpec((B, D), lambda: (0, 0)),
            pl.BlockSpec((B, NUM_VIEWS), lambda: (0, 0)),
            pl.BlockSpec((B, NUM_VIEWS), lambda: (0, 0)),
        ],
        out_specs=[
            pl.BlockSpec((B, D), lambda: (0, 0)),
            pl.BlockSpec((B, D), lambda: (0, 0)),
            pl.BlockSpec((B, NUM_VIEWS, D), lambda: (0, 0, 0)),
            pl.BlockSpec((B, NUM_VIEWS, D), lambda: (0, 0, 0)),
        ],
        out_shape=[
            jax.ShapeDtypeStruct((B, D), jnp.float32),
            jax.ShapeDtypeStruct((B, D), jnp.float32),
            jax.ShapeDtypeStruct((B, NUM_VIEWS, D), jnp.float32),
            jax.ShapeDtypeStruct((B, NUM_VIEWS, D), jnp.float32),
        ],
        interpret=_INTERPRET,
    )(gt, t_pad, t_mask, v0, cx, cy)

    xv = xn_v.reshape(B * NUM_VIEWS, D)

    acts_v = _encode(xv, encoder_v, 2048)
    acts_t = _encode(xn_t, encoder_t, 2048)

    a0 = acts_v.reshape(B, NUM_VIEWS, HID)[:, 0]
    _, sel0 = _topk_latent(a0, 8)
    latent_t, _ = _topk_latent(acts_t, 8)

    recon_f, lat_f, m1, m2 = _decode_v_fused(
        acts_v, sel0, decoder_v_w, decoder_v_b, 2048)

    def _slow():
        lat = _topk_latent(acts_v, 8)[0]
        return _decode(lat, decoder_v_w, decoder_v_b, 2048), lat

    recon_v, latent_v = lax.cond(jnp.all(m2 < m1),
                                 lambda: (recon_f, lat_f), _slow)
    recon_t = _decode(latent_t, decoder_t_w, decoder_t_b, 2048)

    return (recon_v.reshape(B, NUM_VIEWS, D), v_views, recon_t, t_global,
            latent_v.reshape(B, NUM_VIEWS, HID), latent_t)


# speedup vs baseline: 4.2989x; 1.0677x over previous
import functools

import jax
import jax.numpy as jnp
from jax import lax
from jax.experimental import pallas as pl
from jax.experimental.pallas import tpu as pltpu

B = 16
L_PAD = 1024
D = 1024
HID = 16384
TOPK = 32
NUM_VIEWS = 8
GAMMA = 10.0
EPS = 1e-6
T_LEN = 256

ENC_PREC = lax.Precision.DEFAULT
DEC_PREC = lax.Precision.DEFAULT

_INTERPRET = False


def _clipnorm(x, axis):
    n = jnp.sqrt(jnp.sum(x * x, axis=axis, keepdims=True))
    return x / jnp.clip(n, 1e-12)


def _prep_kernel(gt_ref, tp_ref, tm_ref, v0_ref, cx_ref, cy_ref,
                 tg_ref, xnt_ref, vv_ref, xnv_ref):
    tm = tm_ref[...]
    tp = tp_ref[...]
    ts = jnp.sum(tp * tm[:, :, None], axis=1)
    tg = ts / (jnp.sum(tm, axis=1, keepdims=True) + 1e-6)
    tg_ref[...] = tg
    xnt_ref[...] = _clipnorm(tg, -1)

    hg = gt_ref[1].astype(jnp.float32)
    wg = gt_ref[2].astype(jnp.float32)
    x0 = 0.5 / wg
    y0 = 0.5 / hg
    cx = cx_ref[...]
    cy = cy_ref[...]
    dist = (cx - x0) ** 2 + (cy - y0) ** 2
    m = jnp.exp(-GAMMA * dist)
    v0 = v0_ref[...]
    num = m[:, :, None] * v0[:, None, :]
    vv = num / (m + EPS)[:, :, None]
    vv_ref[...] = vv
    xnv_ref[...] = _clipnorm(vv, -1)


def _enc_kernel(x_ref, e_ref, o_ref):
    e = e_ref[...]
    n2 = jnp.sum(e * e, axis=1, keepdims=True)
    w = e / jnp.clip(jnp.sqrt(n2), 1e-12)
    raw = lax.dot_general(x_ref[...], w, (((1,), (1,)), ((), ())),
                          precision=ENC_PREC)
    cos = jnp.clip(raw, -1.0, 1.0)
    o_ref[...] = 2.0 - jnp.sqrt(2.0 - 2.0 * cos)


def _topk_kernel(a_ref, o_ref, s_ref, *, k):
    a = a_ref[...]
    bits = lax.bitcast_convert_type(a, jnp.int32)
    rb = a.shape[0]
    tau = jnp.zeros((rb, 1), jnp.int32)
    for bit in range(30, -1, -1):
        cand = tau | (1 << bit)
        cnt = jnp.sum((bits >= cand).astype(jnp.int32), axis=1, keepdims=True)
        tau = jnp.where(cnt >= k, cand, tau)
    gt = bits > tau
    m = jnp.sum(gt.astype(jnp.int32), axis=1, keepdims=True)
    eq = bits == tau
    need = k - m
    eqcnt = jnp.sum(eq.astype(jnp.int32), axis=1, keepdims=True)

    tie_free = jnp.all(eqcnt == need)

    @pl.when(tie_free)
    def _no_ties():
        sel = gt | eq
        o_ref[...] = jnp.where(sel, a, 0.0)
        s_ref[...] = sel.astype(jnp.float32)

    @pl.when(jnp.logical_not(tie_free))
    def _with_ties():
        iota = lax.broadcasted_iota(jnp.int32, a.shape, 1)
        c = jnp.zeros((rb, 1), jnp.int32)
        for bit in range(13, -1, -1):
            cand = c | (1 << bit)
            cnt = jnp.sum((eq & (iota < cand)).astype(jnp.int32),
                          axis=1, keepdims=True)
            c = jnp.where(cnt < need, cand, c)
        sel = gt | (eq & (iota <= c))
        o_ref[...] = jnp.where(sel, a, 0.0)
        s_ref[...] = sel.astype(jnp.float32)


def _topk_latent(acts, rb):
    r = acts.shape[0]
    return pl.pallas_call(
        functools.partial(_topk_kernel, k=TOPK),
        grid=(r // rb,),
        in_specs=[pl.BlockSpec((rb, HID), lambda i: (i, 0))],
        out_specs=[pl.BlockSpec((rb, HID), lambda i: (i, 0)),
                   pl.BlockSpec((rb, HID), lambda i: (i, 0))],
        out_shape=[jax.ShapeDtypeStruct((r, HID), jnp.float32),
                   jax.ShapeDtypeStruct((r, HID), jnp.float32)],
        interpret=_INTERPRET,
    )(acts)


def _verify_kernel(a_ref, s_ref, l_ref, ok_ref):
    a = a_ref[...]
    s = s_ref[...][0, 0] > 0.5
    sb = jnp.broadcast_to(s[None, :], a.shape)
    m1 = jnp.min(jnp.where(sb, a, 3.0), axis=1, keepdims=True)
    m2 = jnp.max(jnp.where(sb, -1.0, a), axis=1, keepdims=True)
    ok = jnp.all(m2 < m1)
    ok_ref[...] = ok.astype(jnp.float32).reshape(1, 1, 1)
    l_ref[...] = jnp.where(sb, a, 0.0)


def _latent_v_fast(acts_v, sel0):
    sel3 = sel0.reshape(B, 1, HID)
    return pl.pallas_call(
        _verify_kernel,
        grid=(B,),
        in_specs=[
            pl.BlockSpec((NUM_VIEWS, HID), lambda i: (i, 0)),
            pl.BlockSpec((1, 1, HID), lambda i: (i, 0, 0)),
        ],
        out_specs=[
            pl.BlockSpec((NUM_VIEWS, HID), lambda i: (i, 0)),
            pl.BlockSpec((1, 1, 1), lambda i: (i, 0, 0)),
        ],
        out_shape=[
            jax.ShapeDtypeStruct((B * NUM_VIEWS, HID), jnp.float32),
            jax.ShapeDtypeStruct((B, 1, 1), jnp.float32),
        ],
        interpret=_INTERPRET,
    )(acts_v, sel3)


def _dec_kernel(l_ref, d_ref, b_ref, o_ref):
    @pl.when(pl.program_id(0) == 0)
    def _init():
        o_ref[...] = jnp.broadcast_to(b_ref[...], o_ref.shape)
    o_ref[...] += lax.dot_general(l_ref[...], d_ref[...],
                                  (((1,), (1,)), ((), ())),
                                  precision=DEC_PREC)


def _decv_kernel(a_ref, s_ref, d_ref, b_ref, o_ref, l_ref, m1_ref, m2_ref):
    a = a_ref[...]
    s3 = s_ref[...]
    sb = jnp.broadcast_to(s3 > 0.5, (B, NUM_VIEWS, a.shape[1]))
    sb = sb.reshape(B * NUM_VIEWS, a.shape[1])
    lat = jnp.where(sb, a, 0.0)
    l_ref[...] = lat
    m1 = jnp.min(jnp.where(sb, a, 3.0), axis=1, keepdims=True)
    m2 = jnp.max(jnp.where(sb, -1.0, a), axis=1, keepdims=True)

    @pl.when(pl.program_id(0) == 0)
    def _init():
        o_ref[...] = jnp.broadcast_to(b_ref[...], o_ref.shape)
        m1_ref[...] = m1
        m2_ref[...] = m2

    @pl.when(pl.program_id(0) != 0)
    def _acc():
        m1_ref[...] = jnp.minimum(m1_ref[...], m1)
        m2_ref[...] = jnp.maximum(m2_ref[...], m2)

    o_ref[...] += lax.dot_general(lat, d_ref[...],
                                  (((1,), (1,)), ((), ())),
                                  precision=DEC_PREC)


def _decode_v_fused(acts_v, sel0, dec_w, dec_b, hb):
    r = B * NUM_VIEWS
    sel3 = sel0.reshape(B, 1, HID)
    return pl.pallas_call(
        _decv_kernel,
        grid=(HID // hb,),
        in_specs=[
            pl.BlockSpec((r, hb), lambda h: (0, h)),
            pl.BlockSpec((B, 1, hb), lambda h: (0, 0, h)),
            pl.BlockSpec((D, hb), lambda h: (0, h)),
            pl.BlockSpec((1, D), lambda h: (0, 0)),
        ],
        out_specs=[
            pl.BlockSpec((r, D), lambda h: (0, 0)),
            pl.BlockSpec((r, hb), lambda h: (0, h)),
            pl.BlockSpec((r, 1), lambda h: (0, 0)),
            pl.BlockSpec((r, 1), lambda h: (0, 0)),
        ],
        out_shape=[
            jax.ShapeDtypeStruct((r, D), jnp.float32),
            jax.ShapeDtypeStruct((r, HID), jnp.float32),
            jax.ShapeDtypeStruct((r, 1), jnp.float32),
            jax.ShapeDtypeStruct((r, 1), jnp.float32),
        ],
        interpret=_INTERPRET,
    )(acts_v, sel3, dec_w, dec_b.reshape(1, D))


def _encode(x, enc, hb):
    r = x.shape[0]
    return pl.pallas_call(
        _enc_kernel,
        grid=(HID // hb,),
        in_specs=[
            pl.BlockSpec((r, D), lambda h: (0, 0)),
            pl.BlockSpec((hb, D), lambda h: (h, 0)),
        ],
        out_specs=pl.BlockSpec((r, hb), lambda h: (0, h)),
        out_shape=jax.ShapeDtypeStruct((r, HID), jnp.float32),
        interpret=_INTERPRET,
    )(x, enc)


def _decode(latent, dec_w, dec_b, hb):
    r = latent.shape[0]
    return pl.pallas_call(
        _dec_kernel,
        grid=(HID // hb,),
        in_specs=[
            pl.BlockSpec((r, hb), lambda h: (0, h)),
            pl.BlockSpec((D, hb), lambda h: (0, h)),
            pl.BlockSpec((1, D), lambda h: (0, 0)),
        ],
        out_specs=pl.BlockSpec((r, D), lambda h: (0, 0)),
        out_shape=jax.ShapeDtypeStruct((r, D), jnp.float32),
        interpret=_INTERPRET,
    )(latent, dec_w, dec_b.reshape(1, D))


def kernel(v_pad, v_len, grid_thws, t_pad, t_mask, centers,
           encoder_v, decoder_v_w, decoder_v_b,
           encoder_t, decoder_t_w, decoder_t_b):
    del v_len
    v0 = v_pad[:, 0, :]
    cx = centers[:, :, 0]
    cy = centers[:, :, 1]
    gt = grid_thws[0]

    t_global, xn_t, v_views, xn_v = pl.pallas_call(
        _prep_kernel,
        in_specs=[
            pl.BlockSpec(memory_space=pltpu.SMEM),
            pl.BlockSpec((B, T_LEN, D), lambda: (0, 0, 0)),
            pl.BlockSpec((B, T_LEN), lambda: (0, 0)),
            pl.BlockSpec((B, D), lambda: (0, 0)),
            pl.BlockSpec((B, NUM_VIEWS), lambda: (0, 0)),
            pl.BlockSpec((B, NUM_VIEWS), lambda: (0, 0)),
        ],
        out_specs=[
            pl.BlockSpec((B, D), lambda: (0, 0)),
            pl.BlockSpec((B, D), lambda: (0, 0)),
            pl.BlockSpec((B, NUM_VIEWS, D), lambda: (0, 0, 0)),
            pl.BlockSpec((B, NUM_VIEWS, D), lambda: (0, 0, 0)),
        ],
        out_shape=[
            jax.ShapeDtypeStruct((B, D), jnp.float32),
            jax.ShapeDtypeStruct((B, D), jnp.float32),
            jax.ShapeDtypeStruct((B, NUM_VIEWS, D), jnp.float32),
            jax.ShapeDtypeStruct((B, NUM_VIEWS, D), jnp.float32),
        ],
        interpret=_INTERPRET,
    )(gt, t_pad, t_mask, v0, cx, cy)

    xv = xn_v.reshape(B * NUM_VIEWS, D)

    acts_v = _encode(xv, encoder_v, 2048)
    acts_t = _encode(xn_t, encoder_t, 2048)

    a0 = acts_v.reshape(B, NUM_VIEWS, HID)[:, 0]
    _, sel0 = _topk_latent(a0, 8)
    latent_t, _ = _topk_latent(acts_t, 8)

    recon_f, lat_f, m1, m2 = _decode_v_fused(
        acts_v, sel0, decoder_v_w, decoder_v_b, 2048)

    def _slow():
        lat = _topk_latent(acts_v, 8)[0]
        return _decode(lat, decoder_v_w, decoder_v_b, 2048), lat

    recon_v, latent_v = lax.cond(jnp.all(m2 < m1),
                                 lambda: (recon_f, lat_f), _slow)
    recon_t = _decode(latent_t, decoder_t_w, decoder_t_b, 2048)

    return (recon_v.reshape(B, NUM_VIEWS, D), v_views, recon_t, t_global,
            latent_v.reshape(B, NUM_VIEWS, HID), latent_t)


# speedup vs baseline: 4.3007x; 1.0004x over previous
import functools

import jax
import jax.numpy as jnp
from jax import lax
from jax.experimental import pallas as pl
from jax.experimental.pallas import tpu as pltpu

B = 16
L_PAD = 1024
D = 1024
HID = 16384
TOPK = 32
NUM_VIEWS = 8
GAMMA = 10.0
EPS = 1e-6
T_LEN = 256

ENC_PREC = lax.Precision.DEFAULT
DEC_PREC = lax.Precision.DEFAULT

def _clipnorm(x, axis):
    n = jnp.sqrt(jnp.sum(x * x, axis=axis, keepdims=True))
    return x / jnp.clip(n, 1e-12)


def _prep_kernel(gt_ref, tp_ref, tm_ref, v0_ref, cx_ref, cy_ref,
                 tg_ref, xnt_ref, vv_ref, xnv_ref):
    tm = tm_ref[...]
    tp = tp_ref[...]
    ts = jnp.sum(tp * tm[:, :, None], axis=1)
    tg = ts / (jnp.sum(tm, axis=1, keepdims=True) + 1e-6)
    tg_ref[...] = tg
    xnt_ref[...] = _clipnorm(tg, -1)

    hg = gt_ref[1].astype(jnp.float32)
    wg = gt_ref[2].astype(jnp.float32)
    x0 = 0.5 / wg
    y0 = 0.5 / hg
    cx = cx_ref[...]
    cy = cy_ref[...]
    dist = (cx - x0) ** 2 + (cy - y0) ** 2
    m = jnp.exp(-GAMMA * dist)
    v0 = v0_ref[...]
    num = m[:, :, None] * v0[:, None, :]
    vv = num / (m + EPS)[:, :, None]
    vv_ref[...] = vv
    xnv_ref[...] = _clipnorm(vv, -1)


def _enc_kernel(x_ref, e_ref, o_ref):
    e = e_ref[...]
    n2 = jnp.sum(e * e, axis=1, keepdims=True)
    w = e / jnp.clip(jnp.sqrt(n2), 1e-12)
    raw = lax.dot_general(x_ref[...], w, (((1,), (1,)), ((), ())),
                          precision=ENC_PREC)
    cos = jnp.clip(raw, -1.0, 1.0)
    o_ref[...] = 2.0 - jnp.sqrt(2.0 - 2.0 * cos)


def _topk_kernel(a_ref, o_ref, s_ref, *, k):
    a = a_ref[...]
    bits = lax.bitcast_convert_type(a, jnp.int32)
    rb = a.shape[0]
    tau = jnp.zeros((rb, 1), jnp.int32)
    for bit in range(30, -1, -1):
        cand = tau | (1 << bit)
        cnt = jnp.sum((bits >= cand).astype(jnp.int32), axis=1, keepdims=True)
        tau = jnp.where(cnt >= k, cand, tau)
    gt = bits > tau
    m = jnp.sum(gt.astype(jnp.int32), axis=1, keepdims=True)
    eq = bits == tau
    need = k - m
    eqcnt = jnp.sum(eq.astype(jnp.int32), axis=1, keepdims=True)

    tie_free = jnp.all(eqcnt == need)

    @pl.when(tie_free)
    def _no_ties():
        sel = gt | eq
        o_ref[...] = jnp.where(sel, a, 0.0)
        s_ref[...] = sel.astype(jnp.float32)

    @pl.when(jnp.logical_not(tie_free))
    def _with_ties():
        iota = lax.broadcasted_iota(jnp.int32, a.shape, 1)
        c = jnp.zeros((rb, 1), jnp.int32)
        for bit in range(13, -1, -1):
            cand = c | (1 << bit)
            cnt = jnp.sum((eq & (iota < cand)).astype(jnp.int32),
                          axis=1, keepdims=True)
            c = jnp.where(cnt < need, cand, c)
        sel = gt | (eq & (iota <= c))
        o_ref[...] = jnp.where(sel, a, 0.0)
        s_ref[...] = sel.astype(jnp.float32)


def _topk_latent(acts, rb):
    r = acts.shape[0]
    return pl.pallas_call(
        functools.partial(_topk_kernel, k=TOPK),
        grid=(r // rb,),
        in_specs=[pl.BlockSpec((rb, HID), lambda i: (i, 0))],
        out_specs=[pl.BlockSpec((rb, HID), lambda i: (i, 0)),
                   pl.BlockSpec((rb, HID), lambda i: (i, 0))],
        out_shape=[jax.ShapeDtypeStruct((r, HID), jnp.float32),
                   jax.ShapeDtypeStruct((r, HID), jnp.float32)],
    )(acts)


def _dec_kernel(l_ref, d_ref, b_ref, o_ref):
    @pl.when(pl.program_id(0) == 0)
    def _init():
        o_ref[...] = jnp.broadcast_to(b_ref[...], o_ref.shape)
    o_ref[...] += lax.dot_general(l_ref[...], d_ref[...],
                                  (((1,), (1,)), ((), ())),
                                  precision=DEC_PREC)


def _decv_kernel(a_ref, s_ref, d_ref, b_ref, o_ref, l_ref, m1_ref, m2_ref):
    a = a_ref[...]
    s3 = s_ref[...]
    sb = jnp.broadcast_to(s3 > 0.5, (B, NUM_VIEWS, a.shape[1]))
    sb = sb.reshape(B * NUM_VIEWS, a.shape[1])
    lat = jnp.where(sb, a, 0.0)
    l_ref[...] = lat
    m1 = jnp.min(jnp.where(sb, a, 3.0), axis=1, keepdims=True)
    m2 = jnp.max(jnp.where(sb, -1.0, a), axis=1, keepdims=True)

    @pl.when(pl.program_id(0) == 0)
    def _init():
        o_ref[...] = jnp.broadcast_to(b_ref[...], o_ref.shape)
        m1_ref[...] = m1
        m2_ref[...] = m2

    @pl.when(pl.program_id(0) != 0)
    def _acc():
        m1_ref[...] = jnp.minimum(m1_ref[...], m1)
        m2_ref[...] = jnp.maximum(m2_ref[...], m2)

    o_ref[...] += lax.dot_general(lat, d_ref[...],
                                  (((1,), (1,)), ((), ())),
                                  precision=DEC_PREC)


def _decode_v_fused(acts_v, sel0, dec_w, dec_b, hb):
    r = B * NUM_VIEWS
    sel3 = sel0.reshape(B, 1, HID)
    return pl.pallas_call(
        _decv_kernel,
        grid=(HID // hb,),
        in_specs=[
            pl.BlockSpec((r, hb), lambda h: (0, h)),
            pl.BlockSpec((B, 1, hb), lambda h: (0, 0, h)),
            pl.BlockSpec((D, hb), lambda h: (0, h)),
            pl.BlockSpec((1, D), lambda h: (0, 0)),
        ],
        out_specs=[
            pl.BlockSpec((r, D), lambda h: (0, 0)),
            pl.BlockSpec((r, hb), lambda h: (0, h)),
            pl.BlockSpec((r, 1), lambda h: (0, 0)),
            pl.BlockSpec((r, 1), lambda h: (0, 0)),
        ],
        out_shape=[
            jax.ShapeDtypeStruct((r, D), jnp.float32),
            jax.ShapeDtypeStruct((r, HID), jnp.float32),
            jax.ShapeDtypeStruct((r, 1), jnp.float32),
            jax.ShapeDtypeStruct((r, 1), jnp.float32),
        ],
    )(acts_v, sel3, dec_w, dec_b.reshape(1, D))


def _encode(x, enc, hb):
    r = x.shape[0]
    return pl.pallas_call(
        _enc_kernel,
        grid=(HID // hb,),
        in_specs=[
            pl.BlockSpec((r, D), lambda h: (0, 0)),
            pl.BlockSpec((hb, D), lambda h: (h, 0)),
        ],
        out_specs=pl.BlockSpec((r, hb), lambda h: (0, h)),
        out_shape=jax.ShapeDtypeStruct((r, HID), jnp.float32),
    )(x, enc)


def _decode(latent, dec_w, dec_b, hb):
    r = latent.shape[0]
    return pl.pallas_call(
        _dec_kernel,
        grid=(HID // hb,),
        in_specs=[
            pl.BlockSpec((r, hb), lambda h: (0, h)),
            pl.BlockSpec((D, hb), lambda h: (0, h)),
            pl.BlockSpec((1, D), lambda h: (0, 0)),
        ],
        out_specs=pl.BlockSpec((r, D), lambda h: (0, 0)),
        out_shape=jax.ShapeDtypeStruct((r, D), jnp.float32),
    )(latent, dec_w, dec_b.reshape(1, D))


def kernel(v_pad, v_len, grid_thws, t_pad, t_mask, centers,
           encoder_v, decoder_v_w, decoder_v_b,
           encoder_t, decoder_t_w, decoder_t_b):
    del v_len
    v0 = v_pad[:, 0, :]
    cx = centers[:, :, 0]
    cy = centers[:, :, 1]
    gt = grid_thws[0]

    t_global, xn_t, v_views, xn_v = pl.pallas_call(
        _prep_kernel,
        in_specs=[
            pl.BlockSpec(memory_space=pltpu.SMEM),
            pl.BlockSpec((B, T_LEN, D), lambda: (0, 0, 0)),
            pl.BlockSpec((B, T_LEN), lambda: (0, 0)),
            pl.BlockSpec((B, D), lambda: (0, 0)),
            pl.BlockSpec((B, NUM_VIEWS), lambda: (0, 0)),
            pl.BlockSpec((B, NUM_VIEWS), lambda: (0, 0)),
        ],
        out_specs=[
            pl.BlockSpec((B, D), lambda: (0, 0)),
            pl.BlockSpec((B, D), lambda: (0, 0)),
            pl.BlockSpec((B, NUM_VIEWS, D), lambda: (0, 0, 0)),
            pl.BlockSpec((B, NUM_VIEWS, D), lambda: (0, 0, 0)),
        ],
        out_shape=[
            jax.ShapeDtypeStruct((B, D), jnp.float32),
            jax.ShapeDtypeStruct((B, D), jnp.float32),
            jax.ShapeDtypeStruct((B, NUM_VIEWS, D), jnp.float32),
            jax.ShapeDtypeStruct((B, NUM_VIEWS, D), jnp.float32),
        ],
    )(gt, t_pad, t_mask, v0, cx, cy)

    xv = xn_v.reshape(B * NUM_VIEWS, D)

    acts_v = _encode(xv, encoder_v, 2048)
    acts_t = _encode(xn_t, encoder_t, 2048)

    a0 = acts_v.reshape(B, NUM_VIEWS, HID)[:, 0]
    _, sel0 = _topk_latent(a0, 8)
    latent_t, _ = _topk_latent(acts_t, 8)

    recon_f, lat_f, m1, m2 = _decode_v_fused(
        acts_v, sel0, decoder_v_w, decoder_v_b, 2048)

    def _slow():
        lat = _topk_latent(acts_v, 8)[0]
        return _decode(lat, decoder_v_w, decoder_v_b, 2048), lat

    recon_v, latent_v = lax.cond(jnp.all(m2 < m1),
                                 lambda: (recon_f, lat_f), _slow)
    recon_t = _decode(latent_t, decoder_t_w, decoder_t_b, 2048)

    return (recon_v.reshape(B, NUM_VIEWS, D), v_views, recon_t, t_global,
            latent_v.reshape(B, NUM_VIEWS, HID), latent_t)
